# baseline (speedup 1.0000x reference)
import functools

import jax
import jax.numpy as jnp
from jax import lax
from jax.experimental import pallas as pl
from jax.experimental.pallas import tpu as pltpu
from jax.experimental.pallas import tpu_sc as plsc

_B = 32
_H = 12
_L = 577
_C = 768
_NP = _L - 1
_NV = _NP // 16
_DROP = 32
_KEEP_TOTAL = _L - _DROP
_PAD = 640

_LANES = 16


def _vsort(v):
    return plsc.sort_key_val(v, v)[0]


def _merge_lo(a, b):
    rb = lax.rev(b, (0,))
    return _vsort(jnp.minimum(a, rb))


def _merge_both(a, b):
    rb = lax.rev(b, (0,))
    lo = jnp.minimum(a, rb)
    hi = jnp.maximum(a, rb)
    return _vsort(lo), _vsort(hi)


def _copy_body(x_ref, o_ref):
    o_ref[...] = x_ref[...]


def _tc_copy(x):
    return pl.pallas_call(
        _copy_body,
        grid=(_B // 4,),
        in_specs=[pl.BlockSpec((4, _L, _C), lambda i: (i, 0, 0))],
        out_specs=pl.BlockSpec((4, _L, _C), lambda i: (i, 0, 0)),
        out_shape=jax.ShapeDtypeStruct((_B, _L, _C), jnp.float32),
    )(x)


def _mean_body(attn_ref, cls_ref, pad_ref):
    for j in range(8):
        acc = attn_ref[_H * j, 0, :]
        for h in range(1, _H):
            acc = acc + attn_ref[_H * j + h, 0, :]
        m = acc * jnp.float32(1.0 / _H)
        cls_ref[j, :] = m[1:]
        pad_ref[j, pl.ds(0, _NP)] = m[1:]
        pad_ref[j, pl.ds(_NP, _PAD - _NP)] = jnp.zeros(
            (_PAD - _NP,), jnp.float32
        )


def _tc_mean(attn):
    attn3 = attn.reshape(_B * _H, _L, _L)
    return pl.pallas_call(
        _mean_body,
        grid=(_B // 8,),
        in_specs=[pl.BlockSpec((8 * _H, 8, _L), lambda i: (i, 0, 0))],
        out_specs=[
            pl.BlockSpec((8, _NP), lambda i: (i, 0)),
            pl.BlockSpec((8, _PAD), lambda i: (i, 0)),
        ],
        out_shape=[
            jax.ShapeDtypeStruct((_B, _NP), jnp.float32),
            jax.ShapeDtypeStruct((_B, _PAD), jnp.float32),
        ],
    )(attn3)


def _sc_select(cls_pad):
    mesh = plsc.VectorSubcoreMesh(
        core_axis_name="c", subcore_axis_name="s", num_cores=2, num_subcores=16
    )

    @functools.partial(
        pl.kernel,
        out_type=jax.ShapeDtypeStruct((_B, _PAD), jnp.int32),
        mesh=mesh,
        compiler_params=pltpu.CompilerParams(needs_layout_passes=False),
        scratch_types=[
            pltpu.VMEM((_PAD,), jnp.float32),
            pltpu.VMEM((_PAD,), jnp.int32),
            pltpu.SemaphoreType.DMA,
        ],
    )
    def k(cls_hbm, idx_hbm, cls_v, idx_v, sem):
        b = lax.axis_index("s") * 2 + lax.axis_index("c")
        pltpu.async_copy(cls_hbm.at[b], cls_v, sem).wait()

        vals = [cls_v[pl.ds(16 * i, 16)] for i in range(_NV)]

        s0, s1 = _merge_both(_vsort(vals[0]), _vsort(vals[1]))
        for i in range(2, _NV):
            c_sorted = _vsort(vals[i])
            m_lo = _merge_lo(s1, c_sorted)
            s0, s1 = _merge_both(s0, m_lo)
        t = jnp.max(s1)
        t_vec = jnp.broadcast_to(t, (_LANES,))

        lt_acc = jnp.zeros((_LANES,), jnp.int32)
        eq_acc = jnp.zeros((_LANES,), jnp.int32)
        one = jnp.ones((_LANES,), jnp.int32)
        zero = jnp.zeros((_LANES,), jnp.int32)
        for i in range(_NV):
            v = vals[i]
            lt_acc = lt_acc + jnp.where(v < t_vec, one, zero)
            eq_acc = eq_acc + jnp.where(v == t_vec, one, zero)
        c_lt = jnp.sum(lt_acc)
        m_eq = jnp.sum(eq_acc)
        keep_ties = m_eq - (jnp.int32(_DROP) - c_lt)
        keep_ties_vec = jnp.broadcast_to(keep_ties, (_LANES,))

        idx_v[pl.ds(0, 16)] = zero
        pos = jnp.int32(1)
        q_run = jnp.int32(0)
        base_iota = lax.iota(jnp.int32, _LANES)
        for i in range(_NV):
            v = vals[i]
            eq = v == t_vec
            eq_i32 = jnp.where(eq, one, zero)
            cum = plsc.cumsum(eq_i32) + jnp.broadcast_to(q_run, (_LANES,))
            kept = (v > t_vec) | (eq & (cum <= keep_ties_vec))
            idx = base_iota + jnp.int32(16 * i + 1)
            plsc.store_compressed(idx_v.at[pl.ds(pos, 16)], idx, mask=kept)
            pos = pos + jnp.sum(jnp.where(kept, one, zero))
            q_run = q_run + jnp.sum(eq_i32)

        pltpu.sync_copy(idx_v, idx_hbm.at[b])

    return k(cls_pad)


def _pack_body(idx_ref, out_ref):
    out_ref[...] = idx_ref[:, : _KEEP_TOTAL]


def _tc_pack(idx_pad):
    return pl.pallas_call(
        _pack_body,
        grid=(_B // 8,),
        in_specs=[pl.BlockSpec((8, _PAD), lambda i: (i, 0))],
        out_specs=pl.BlockSpec((8, _KEEP_TOTAL), lambda i: (i, 0)),
        out_shape=jax.ShapeDtypeStruct((_B, _KEEP_TOTAL), jnp.int32),
    )(idx_pad)


def kernel(x, attn):
    cls_attn, cls_pad = _tc_mean(attn)
    idx_pad = _sc_select(cls_pad)
    full_indices = _tc_pack(idx_pad)
    return (_tc_copy(x), full_indices, cls_attn, _L - _DROP)


# speedup vs baseline: 9.2904x; 9.2904x over previous
import functools

import jax
import jax.numpy as jnp
from jax import lax
from jax.experimental import pallas as pl
from jax.experimental.pallas import tpu as pltpu
from jax.experimental.pallas import tpu_sc as plsc

_B = 32
_H = 12
_L = 577
_C = 768
_NP = _L - 1
_NV = _NP // 16
_DROP = 32
_KEEP_TOTAL = _L - _DROP
_PAD = 640

_LANES = 16


def _vsort(v):
    return plsc.sort_key_val(v, v)[0]


def _merge_lo(a, b):
    rb = lax.rev(b, (0,))
    return _vsort(jnp.minimum(a, rb))


def _merge_both(a, b):
    rb = lax.rev(b, (0,))
    lo = jnp.minimum(a, rb)
    hi = jnp.maximum(a, rb)
    return _vsort(lo), _vsort(hi)


def _mean_body(attn_ref, cls_ref, pad_ref):
    acc = attn_ref[0, 0, :, :]
    for h in range(1, _H):
        acc = acc + attn_ref[h, 0, :, :]
    m = acc * jnp.float32(1.0 / _H)
    cls_ref[...] = m[:, 1:]
    pad_ref[:, pl.ds(0, _NP)] = m[:, 1:]
    pad_ref[:, pl.ds(_NP, _PAD - _NP)] = jnp.zeros((_B, _PAD - _NP),
                                                   jnp.float32)


def _tc_mean(attn):
    attn_t = jnp.transpose(attn, (1, 2, 0, 3))
    return pl.pallas_call(
        _mean_body,
        grid=(1,),
        in_specs=[pl.BlockSpec((_H, 1, _B, _L), lambda i: (0, 0, 0, 0))],
        out_specs=[
            pl.BlockSpec((_B, _NP), lambda i: (0, 0)),
            pl.BlockSpec((_B, _PAD), lambda i: (0, 0)),
        ],
        out_shape=[
            jax.ShapeDtypeStruct((_B, _NP), jnp.float32),
            jax.ShapeDtypeStruct((_B, _PAD), jnp.float32),
        ],
    )(attn_t)


def _sc_select(cls_pad):
    mesh = plsc.VectorSubcoreMesh(
        core_axis_name="c", subcore_axis_name="s", num_cores=2, num_subcores=16
    )

    @functools.partial(
        pl.kernel,
        out_type=jax.ShapeDtypeStruct((_B, _PAD), jnp.int32),
        mesh=mesh,
        compiler_params=pltpu.CompilerParams(needs_layout_passes=False),
        scratch_types=[
            pltpu.VMEM((_PAD,), jnp.float32),
            pltpu.VMEM((_PAD,), jnp.int32),
            pltpu.SemaphoreType.DMA,
        ],
    )
    def k(cls_hbm, idx_hbm, cls_v, idx_v, sem):
        b = lax.axis_index("s") * 2 + lax.axis_index("c")
        pltpu.async_copy(cls_hbm.at[b], cls_v, sem).wait()

        vals = [cls_v[pl.ds(16 * i, 16)] for i in range(_NV)]

        s0, s1 = _merge_both(_vsort(vals[0]), _vsort(vals[1]))
        for i in range(2, _NV):
            c_sorted = _vsort(vals[i])
            m_lo = _merge_lo(s1, c_sorted)
            s0, s1 = _merge_both(s0, m_lo)
        t = jnp.max(s1)
        t_vec = jnp.broadcast_to(t, (_LANES,))

        lt_acc = jnp.zeros((_LANES,), jnp.int32)
        eq_acc = jnp.zeros((_LANES,), jnp.int32)
        one = jnp.ones((_LANES,), jnp.int32)
        zero = jnp.zeros((_LANES,), jnp.int32)
        for i in range(_NV):
            v = vals[i]
            lt_acc = lt_acc + jnp.where(v < t_vec, one, zero)
            eq_acc = eq_acc + jnp.where(v == t_vec, one, zero)
        c_lt = jnp.sum(lt_acc)
        m_eq = jnp.sum(eq_acc)
        keep_ties = m_eq - (jnp.int32(_DROP) - c_lt)
        keep_ties_vec = jnp.broadcast_to(keep_ties, (_LANES,))

        idx_v[pl.ds(0, 16)] = zero
        pos = jnp.int32(1)
        q_run = jnp.int32(0)
        base_iota = lax.iota(jnp.int32, _LANES)
        for i in range(_NV):
            v = vals[i]
            eq = v == t_vec
            eq_i32 = jnp.where(eq, one, zero)
            cum = plsc.cumsum(eq_i32) + jnp.broadcast_to(q_run, (_LANES,))
            kept = (v > t_vec) | (eq & (cum <= keep_ties_vec))
            idx = base_iota + jnp.int32(16 * i + 1)
            plsc.store_compressed(idx_v.at[pl.ds(pos, 16)], idx, mask=kept)
            pos = pos + jnp.sum(jnp.where(kept, one, zero))
            q_run = q_run + jnp.sum(eq_i32)

        pltpu.sync_copy(idx_v, idx_hbm.at[b])

    return k(cls_pad)


def _pack_body(idx_ref, out_ref):
    out_ref[...] = idx_ref[:, : _KEEP_TOTAL]


def _tc_pack(idx_pad):
    return pl.pallas_call(
        _pack_body,
        grid=(_B // 8,),
        in_specs=[pl.BlockSpec((8, _PAD), lambda i: (i, 0))],
        out_specs=pl.BlockSpec((8, _KEEP_TOTAL), lambda i: (i, 0)),
        out_shape=jax.ShapeDtypeStruct((_B, _KEEP_TOTAL), jnp.int32),
    )(idx_pad)


def kernel(x, attn):
    cls_attn, cls_pad = _tc_mean(attn)
    idx_pad = _sc_select(cls_pad)
    full_indices = _tc_pack(idx_pad)
    return (x, full_indices, cls_attn, _L - _DROP)
